# baseline (speedup 1.0000x reference)
import functools

import jax
import jax.numpy as jnp
from jax.experimental import pallas as pl
from jax.experimental.pallas import tpu as pltpu

N = 10000
E = 160000
D = 256
M = 4096
RADIUS = 3

BN = 400
NBLK = N // BN


def _main_body(feat_a_ref, pm_ref, z1_ref, z2_ref, z4_ref,
               feat_b_ref, w_fuse_t_ref, w_prev_t_ref,
               w1_t_ref, w2_t_ref, w3_t_ref, bias_ref,
               r_ref, psum_ref, psq_ref, fw_ref):
    i = pl.program_id(0)

    @pl.when(i == 0)
    def _():
        fw_ref[...] = jnp.dot(feat_b_ref[...], w_fuse_t_ref[...],
                              preferred_element_type=jnp.float32)

    acc = jnp.dot(feat_a_ref[...], w_prev_t_ref[...],
                  preferred_element_type=jnp.float32)
    acc += jnp.dot(z1_ref[...], w1_t_ref[...],
                   preferred_element_type=jnp.float32)
    acc += jnp.dot(z2_ref[...], w2_t_ref[...],
                   preferred_element_type=jnp.float32)
    acc += jnp.dot(z4_ref[...], w3_t_ref[...],
                   preferred_element_type=jnp.float32)
    acc += jnp.dot(pm_ref[...], fw_ref[...],
                   preferred_element_type=jnp.float32)
    acc += bias_ref[...]

    col = jax.lax.broadcasted_iota(jnp.int32, (BN, D), 1)
    acc = jnp.where(col >= D // 2, jnp.maximum(acc, 0.0), acc)

    r_ref[...] = acc
    psum_ref[...] = jnp.broadcast_to(jnp.sum(acc, axis=0, keepdims=True) / 8.0,
                                     (8, D))
    psq_ref[...] = jnp.broadcast_to(jnp.sum(acc * acc, axis=0, keepdims=True) / 8.0,
                                    (8, D))


def _bn_body(r_ref, psum_ref, psq_ref, gamma_ref, beta_ref, out_ref):
    mean = jnp.sum(psum_ref[...], axis=0, keepdims=True) / N
    var = jnp.sum(psq_ref[...], axis=0, keepdims=True) / N - mean * mean
    scale = jax.lax.rsqrt(var + 1e-5) * gamma_ref[...]
    out_ref[...] = (r_ref[...] - mean) * scale + beta_ref[...]


def _dense_part(feat_a, pm_pd, z1, z2, z4, feat_b, W_prev, W_rad,
                W_fuse, bias, bn_gamma, bn_beta):
    r, psum, psq = pl.pallas_call(
        _main_body,
        grid=(NBLK,),
        in_specs=[
            pl.BlockSpec((BN, D), lambda i: (i, 0)),
            pl.BlockSpec((BN, M), lambda i: (i, 0)),
            pl.BlockSpec((BN, D), lambda i: (i, 0)),
            pl.BlockSpec((BN, D), lambda i: (i, 0)),
            pl.BlockSpec((BN, D), lambda i: (i, 0)),
            pl.BlockSpec((M, D), lambda i: (0, 0)),
            pl.BlockSpec((D, D), lambda i: (0, 0)),
            pl.BlockSpec((D, D), lambda i: (0, 0)),
            pl.BlockSpec((D, D), lambda i: (0, 0)),
            pl.BlockSpec((D, D), lambda i: (0, 0)),
            pl.BlockSpec((D, D), lambda i: (0, 0)),
            pl.BlockSpec((1, D), lambda i: (0, 0)),
        ],
        out_specs=[
            pl.BlockSpec((BN, D), lambda i: (i, 0)),
            pl.BlockSpec((8, D), lambda i: (i, 0)),
            pl.BlockSpec((8, D), lambda i: (i, 0)),
        ],
        out_shape=[
            jax.ShapeDtypeStruct((N, D), jnp.float32),
            jax.ShapeDtypeStruct((NBLK * 8, D), jnp.float32),
            jax.ShapeDtypeStruct((NBLK * 8, D), jnp.float32),
        ],
        scratch_shapes=[pltpu.VMEM((M, D), jnp.float32)],
    )(feat_a, pm_pd, z1, z2, z4, feat_b,
      W_fuse.T, W_prev.T, W_rad[0].T, W_rad[1].T, W_rad[2].T,
      bias.reshape(1, D))

    out = pl.pallas_call(
        _bn_body,
        grid=(NBLK,),
        in_specs=[
            pl.BlockSpec((BN, D), lambda i: (i, 0)),
            pl.BlockSpec((NBLK * 8, D), lambda i: (0, 0)),
            pl.BlockSpec((NBLK * 8, D), lambda i: (0, 0)),
            pl.BlockSpec((1, D), lambda i: (0, 0)),
            pl.BlockSpec((1, D), lambda i: (0, 0)),
        ],
        out_specs=pl.BlockSpec((BN, D), lambda i: (i, 0)),
        out_shape=jax.ShapeDtypeStruct((N, D), jnp.float32),
    )(r, psum, psq, bn_gamma.reshape(1, D), bn_beta.reshape(1, D))
    return out


def kernel(feat_a, feat_b, deg, pm_pd, edge_index,
           W_prev, b_prev, W_deg, b_deg, W_rad, b_rad,
           W_fuse, b_fuse, bn_gamma, bn_beta):
    src = edge_index[0]
    dst = edge_index[1]
    z1 = jax.ops.segment_sum(feat_a[src], dst, num_segments=N)
    z2 = jax.ops.segment_sum(z1[src], dst, num_segments=N)
    z3 = jax.ops.segment_sum(z2[src], dst, num_segments=N)
    z4 = jax.ops.segment_sum(z3[src], dst, num_segments=N)

    bias = b_prev + b_rad[0] + b_rad[1] + b_rad[2] + b_fuse
    return _dense_part(feat_a, pm_pd, z1, z2, z4, feat_b,
                       W_prev, W_rad, W_fuse, bias, bn_gamma, bn_beta)


# speedup vs baseline: 3.6655x; 3.6655x over previous
import functools

import jax
import jax.numpy as jnp
from jax import lax
from jax.experimental import pallas as pl
from jax.experimental.pallas import tpu as pltpu
from jax.experimental.pallas import tpu_sc as plsc

N = 10000
E = 160000
D = 256
M = 4096
H = D // 2

BN = 400
NBLK = N // BN

NTILE = 16
EPT = E // NTILE
B = 128
NIT = 80
EPAD = NIT * B
CH = 40
ACCROWS = N + 8
RPT_A = 632
RPT_B = N - 15 * RPT_A

_mesh = plsc.VectorSubcoreMesh(core_axis_name="c", subcore_axis_name="s")


@functools.partial(
    pl.kernel,
    out_type=jax.ShapeDtypeStruct((2, N, H), jnp.float32),
    mesh=_mesh,
    scratch_types=[
        pltpu.VMEM((CH, B), jnp.int32),
        pltpu.VMEM((CH, B), jnp.int32),
        pltpu.VMEM((B, H), jnp.float32),
        pltpu.VMEM((B, H), jnp.float32),
        pltpu.VMEM_SHARED((ACCROWS, H), jnp.float32),
        pltpu.SemaphoreType.DMA,
        pltpu.SemaphoreType.DMA,
    ],
)
def _hop(z_hbm, src_hbm, dst_hbm, zero_hbm, out_hbm,
         src_v, dst_v, buf0, buf1, acc, sem0, sem1):
    c = lax.axis_index("c")
    w = lax.axis_index("s")

    @pl.when(w < 15)
    def _():
        pltpu.sync_copy(zero_hbm.at[pl.ds(w * RPT_A, RPT_A)],
                        acc.at[pl.ds(w * RPT_A, RPT_A)])

    @pl.when(w == 15)
    def _():
        pltpu.sync_copy(zero_hbm.at[pl.ds(15 * RPT_A, RPT_B)],
                        acc.at[pl.ds(15 * RPT_A, RPT_B)])

    plsc.subcore_barrier()

    zc = z_hbm.at[c]

    for ch in range(NIT // CH):
        pltpu.sync_copy(src_hbm.at[w].at[pl.ds(ch * CH, CH)], src_v)
        pltpu.sync_copy(dst_hbm.at[w].at[pl.ds(ch * CH, CH)], dst_v)
        pltpu.async_copy(zc.at[src_v.at[0]], buf0, sem0)

        def pair(k, carry):
            j0 = 2 * k
            pltpu.async_copy(zc.at[src_v.at[j0 + 1]], buf1, sem1)
            pltpu.make_async_copy(zc.at[src_v.at[j0]], buf0, sem0).wait()
            pltpu.sync_copy(buf0, acc.at[dst_v.at[j0]], add=True)

            @pl.when(j0 + 2 < CH)
            def _():
                pltpu.async_copy(zc.at[src_v.at[j0 + 2]], buf0, sem0)

            pltpu.make_async_copy(zc.at[src_v.at[j0 + 1]], buf1, sem1).wait()
            pltpu.sync_copy(buf1, acc.at[dst_v.at[j0 + 1]], add=True)
            return carry

        lax.fori_loop(0, CH // 2, pair, 0)

    plsc.subcore_barrier()

    @pl.when(w < 15)
    def _():
        pltpu.sync_copy(acc.at[pl.ds(w * RPT_A, RPT_A)],
                        out_hbm.at[c].at[pl.ds(w * RPT_A, RPT_A)])

    @pl.when(w == 15)
    def _():
        pltpu.sync_copy(acc.at[pl.ds(15 * RPT_A, RPT_B)],
                        out_hbm.at[c].at[pl.ds(15 * RPT_A, RPT_B)])


def _sc_hops(feat_a, edge_index):
    src = edge_index[0].reshape(NTILE, EPT)
    dst = edge_index[1].reshape(NTILE, EPT)
    pad = EPAD - EPT
    srcp = jnp.concatenate(
        [src, jnp.zeros((NTILE, pad), jnp.int32)], axis=1).reshape(NTILE, NIT, B)
    dstp = jnp.concatenate(
        [dst, jnp.full((NTILE, pad), N, jnp.int32)], axis=1).reshape(NTILE, NIT, B)
    zeros = jnp.zeros((N, H), jnp.float32)

    z = feat_a.reshape(N, 2, H).transpose(1, 0, 2)
    zs = []
    for _ in range(4):
        z = _hop(z, srcp, dstp, zeros)
        zs.append(z)

    def back(zq):
        return zq.transpose(1, 0, 2).reshape(N, D)

    return back(zs[0]), back(zs[1]), back(zs[3])


def _main_body(feat_a_ref, pm_ref, z1_ref, z2_ref, z4_ref,
               feat_b_ref, w_fuse_t_ref, w_prev_t_ref,
               w1_t_ref, w2_t_ref, w3_t_ref, bias_ref,
               r_ref, psum_ref, psq_ref, fw_ref):
    i = pl.program_id(0)

    @pl.when(i == 0)
    def _():
        fw_ref[...] = jnp.dot(feat_b_ref[...], w_fuse_t_ref[...],
                              preferred_element_type=jnp.float32)

    acc = jnp.dot(feat_a_ref[...], w_prev_t_ref[...],
                  preferred_element_type=jnp.float32)
    acc += jnp.dot(z1_ref[...], w1_t_ref[...],
                   preferred_element_type=jnp.float32)
    acc += jnp.dot(z2_ref[...], w2_t_ref[...],
                   preferred_element_type=jnp.float32)
    acc += jnp.dot(z4_ref[...], w3_t_ref[...],
                   preferred_element_type=jnp.float32)
    acc += jnp.dot(pm_ref[...], fw_ref[...],
                   preferred_element_type=jnp.float32)
    acc += bias_ref[...]

    col = jax.lax.broadcasted_iota(jnp.int32, (BN, D), 1)
    acc = jnp.where(col >= D // 2, jnp.maximum(acc, 0.0), acc)

    r_ref[...] = acc
    psum_ref[...] = jnp.broadcast_to(jnp.sum(acc, axis=0, keepdims=True) / 8.0,
                                     (8, D))
    psq_ref[...] = jnp.broadcast_to(jnp.sum(acc * acc, axis=0, keepdims=True) / 8.0,
                                    (8, D))


def _bn_body(r_ref, psum_ref, psq_ref, gamma_ref, beta_ref, out_ref):
    mean = jnp.sum(psum_ref[...], axis=0, keepdims=True) / N
    var = jnp.sum(psq_ref[...], axis=0, keepdims=True) / N - mean * mean
    scale = jax.lax.rsqrt(var + 1e-5) * gamma_ref[...]
    out_ref[...] = (r_ref[...] - mean) * scale + beta_ref[...]


def _dense_part(feat_a, pm_pd, z1, z2, z4, feat_b, W_prev, W_rad,
                W_fuse, bias, bn_gamma, bn_beta):
    r, psum, psq = pl.pallas_call(
        _main_body,
        grid=(NBLK,),
        in_specs=[
            pl.BlockSpec((BN, D), lambda i: (i, 0)),
            pl.BlockSpec((BN, M), lambda i: (i, 0)),
            pl.BlockSpec((BN, D), lambda i: (i, 0)),
            pl.BlockSpec((BN, D), lambda i: (i, 0)),
            pl.BlockSpec((BN, D), lambda i: (i, 0)),
            pl.BlockSpec((M, D), lambda i: (0, 0)),
            pl.BlockSpec((D, D), lambda i: (0, 0)),
            pl.BlockSpec((D, D), lambda i: (0, 0)),
            pl.BlockSpec((D, D), lambda i: (0, 0)),
            pl.BlockSpec((D, D), lambda i: (0, 0)),
            pl.BlockSpec((D, D), lambda i: (0, 0)),
            pl.BlockSpec((1, D), lambda i: (0, 0)),
        ],
        out_specs=[
            pl.BlockSpec((BN, D), lambda i: (i, 0)),
            pl.BlockSpec((8, D), lambda i: (i, 0)),
            pl.BlockSpec((8, D), lambda i: (i, 0)),
        ],
        out_shape=[
            jax.ShapeDtypeStruct((N, D), jnp.float32),
            jax.ShapeDtypeStruct((NBLK * 8, D), jnp.float32),
            jax.ShapeDtypeStruct((NBLK * 8, D), jnp.float32),
        ],
        scratch_shapes=[pltpu.VMEM((M, D), jnp.float32)],
    )(feat_a, pm_pd, z1, z2, z4, feat_b,
      W_fuse.T, W_prev.T, W_rad[0].T, W_rad[1].T, W_rad[2].T,
      bias.reshape(1, D))

    out = pl.pallas_call(
        _bn_body,
        grid=(NBLK,),
        in_specs=[
            pl.BlockSpec((BN, D), lambda i: (i, 0)),
            pl.BlockSpec((NBLK * 8, D), lambda i: (0, 0)),
            pl.BlockSpec((NBLK * 8, D), lambda i: (0, 0)),
            pl.BlockSpec((1, D), lambda i: (0, 0)),
            pl.BlockSpec((1, D), lambda i: (0, 0)),
        ],
        out_specs=pl.BlockSpec((BN, D), lambda i: (i, 0)),
        out_shape=jax.ShapeDtypeStruct((N, D), jnp.float32),
    )(r, psum, psq, bn_gamma.reshape(1, D), bn_beta.reshape(1, D))
    return out


def kernel(feat_a, feat_b, deg, pm_pd, edge_index,
           W_prev, b_prev, W_deg, b_deg, W_rad, b_rad,
           W_fuse, b_fuse, bn_gamma, bn_beta):
    z1, z2, z4 = _sc_hops(feat_a, edge_index)
    bias = b_prev + b_rad[0] + b_rad[1] + b_rad[2] + b_fuse
    return _dense_part(feat_a, pm_pd, z1, z2, z4, feat_b,
                       W_prev, W_rad, W_fuse, bias, bn_gamma, bn_beta)
